# baseline (speedup 1.0000x reference)
import functools
import jax
import jax.numpy as jnp
from jax import lax
from jax.experimental import pallas as pl
from jax.experimental.pallas import tpu as pltpu
from jax.experimental.pallas import tpu_sc as plsc

_EXPR = 80
_L = 64
_C = 32
_NPAD = 72
_NW = 32
_CHUNK = 32
_B2 = 2048


def _premix_body(E_ref, flx_ref, fly_ref, flz_ref, v0_ref, g0_ref,
                 mx_ref, my_ref, mz_ref):
    f32 = jnp.float32
    v0 = v0_ref[...]
    W0 = v0 * (g0_ref[...] * lax.rsqrt(jnp.sum(v0 * v0, axis=1, keepdims=True)))
    E = E_ref[...]
    for a, (fl_ref, m_ref) in enumerate(
            ((flx_ref, mx_ref), (fly_ref, my_ref), (flz_ref, mz_ref))):
        tab = jnp.dot(fl_ref[...], E, preferred_element_type=f32)
        W0a = jnp.concatenate(
            [W0[:, _C * a:_C * a + _C],
             W0[:, 3 * _C + _C * a:3 * _C + _C * a + _C]], axis=1)
        Ma = lax.dot_general(tab, W0a, (((1,), (1,)), ((), ())),
                             preferred_element_type=f32)
        m_ref[0:_L, :] = Ma
        m_ref[_L:_NPAD, :] = jnp.zeros((_NPAD - _L, 128), f32)


def _sc_body(mx_hbm, my_hbm, mz_hbm, x_hbm, y_hbm, z_hbm, out_hbm,
             xb, yb, zb, rxl, rxr, ryl, ryr, rzl, rzr, h0b, sem):
    npts = x_hbm.shape[0]
    pw = npts // _NW
    nchunks = pw // _CHUNK
    wid = lax.axis_index("s") * 2 + lax.axis_index("c")
    base = wid * pw

    lane = lax.iota(jnp.int32, 16)
    dn = lax.GatherDimensionNumbers(
        offset_dims=(), collapsed_slice_dims=(0,), start_index_map=(0,))

    def splat(vec, j):
        idx = jnp.reshape(lane * 0 + j, (16, 1))
        return lax.gather(vec, idx, dn, (1,),
                          mode=lax.GatherScatterMode.PROMISE_IN_BOUNDS)

    axes = ((xb, mx_hbm, rxl, rxr), (yb, my_hbm, ryl, ryr), (zb, mz_hbm, rzl, rzr))

    def chunk_body(ci, carry):
        cbase = base + ci * _CHUNK
        pltpu.sync_copy(x_hbm.at[pl.ds(cbase, _CHUNK)], xb)
        pltpu.sync_copy(y_hbm.at[pl.ds(cbase, _CHUNK)], yb)
        pltpu.sync_copy(z_hbm.at[pl.ds(cbase, _CHUNK)], zb)

        ws = []
        handles = []
        for g in range(_CHUNK // 16):
            for buf, tab, rl_ref, rr_ref in axes:
                p = buf[pl.ds(g * 16, 16)]
                p = jnp.minimum(jnp.maximum(p, 0.0), 1.0) * (_L - 1.0)
                li = p.astype(jnp.int32)
                ws.append(p - li.astype(jnp.float32))
                dst_l = rl_ref.at[pl.ds(g * 16, 16), :]
                dst_r = rr_ref.at[pl.ds(g * 16, 16), :]
                handles.append(pltpu.async_copy(tab.at[li], dst_l, sem))
                handles.append(pltpu.async_copy(tab.at[li + 1], dst_r, sem))
        for h in handles:
            h.wait()

        for g in range(_CHUNK // 16):
            for j in range(16):
                pt = g * 16 + j
                acc = [None] * 8
                for a in range(3):
                    _, _, rl_ref, rr_ref = axes[a]
                    wv = splat(ws[g * 3 + a], j)
                    for k in range(8):
                        rl = rl_ref[pt, pl.ds(16 * k, 16)]
                        rr = rr_ref[pt, pl.ds(16 * k, 16)]
                        c = rl + wv * (rr - rl)
                        acc[k] = c if a == 0 else acc[k] + c
                hstart = pt * 128
                for k in range(8):
                    h0b[pl.ds(hstart + 16 * k, 16)] = acc[k]
        pltpu.sync_copy(h0b, out_hbm.at[pl.ds(cbase * 128, _CHUNK * 128)])
        return carry

    lax.fori_loop(0, nchunks, chunk_body, 0, unroll=False)


def _mlp_body(h0_ref, b0_ref, v1_ref, g1_ref, b1_ref, v2_ref, g2_ref, b2_ref,
              o_ref):
    f32 = jnp.float32
    v1 = v1_ref[...]
    W1 = v1 * (g1_ref[...] * lax.rsqrt(jnp.sum(v1 * v1, axis=1, keepdims=True)))
    v2 = v2_ref[...]
    W2 = v2 * (g2_ref[...] * lax.rsqrt(jnp.sum(v2 * v2, axis=1, keepdims=True)))
    a0 = jnp.maximum(h0_ref[...] + b0_ref[...], 0.0)
    h1 = lax.dot_general(a0, W1, (((1,), (1,)), ((), ())),
                         preferred_element_type=f32) + b1_ref[...]
    h1 = jnp.maximum(h1, 0.0)
    o_ref[...] = lax.dot_general(W2, h1, (((1,), (1,)), ((), ())),
                                 preferred_element_type=f32) + b2_ref[...]


@jax.jit
def kernel(expr, jaw_quat_weight, xyz, feat_lines_x, feat_lines_y,
           feat_lines_z, v0, g0, b0, v1, g1, b1, v2, g2, b2):
    f32 = jnp.float32
    n = xyz.shape[0]
    e = expr.reshape(-1)[:_EXPR]
    jw = jaw_quat_weight.reshape(-1)
    eye = jnp.eye(_C, dtype=f32)
    Ebs = (e[:, None, None] * eye).reshape(_EXPR * _C, _C)
    Ejw = (jw[:, None, None] * eye).reshape(jw.shape[0] * _C, _C)
    E = jnp.concatenate([
        jnp.concatenate([Ebs, jnp.zeros_like(Ebs)], axis=1),
        jnp.concatenate([jnp.zeros_like(Ejw), Ejw], axis=1)], axis=0)

    def lines2d(fl):
        return fl.transpose(1, 0, 2).reshape(_L, fl.shape[0] * _C)

    whole = lambda shp: pl.BlockSpec(shp, lambda *_: (0,) * len(shp))

    mx, my, mz = pl.pallas_call(
        _premix_body,
        grid=(1,),
        in_specs=[whole(E.shape), whole((_L, 96 * _C)), whole((_L, 96 * _C)),
                  whole((_L, 96 * _C)), whole(v0.shape), whole((128, 1))],
        out_specs=[whole((_NPAD, 128))] * 3,
        out_shape=[jax.ShapeDtypeStruct((_NPAD, 128), f32)] * 3,
    )(E, lines2d(feat_lines_x), lines2d(feat_lines_y), lines2d(feat_lines_z),
      v0, g0.reshape(-1, 1))

    xyzT = xyz.T
    mesh = plsc.VectorSubcoreMesh(core_axis_name="c", subcore_axis_name="s")
    sc = pl.kernel(
        _sc_body, mesh=mesh,
        out_type=jax.ShapeDtypeStruct((n * 128,), f32),
        scratch_types=[
            pltpu.VMEM((_CHUNK,), f32),
            pltpu.VMEM((_CHUNK,), f32),
            pltpu.VMEM((_CHUNK,), f32),
            pltpu.VMEM((_CHUNK, 128), f32),
            pltpu.VMEM((_CHUNK, 128), f32),
            pltpu.VMEM((_CHUNK, 128), f32),
            pltpu.VMEM((_CHUNK, 128), f32),
            pltpu.VMEM((_CHUNK, 128), f32),
            pltpu.VMEM((_CHUNK, 128), f32),
            pltpu.VMEM((_CHUNK * 128,), f32),
            pltpu.SemaphoreType.DMA,
        ],
    )
    h0 = sc(mx, my, mz, xyzT[0], xyzT[1], xyzT[2]).reshape(n, 128)

    out = pl.pallas_call(
        _mlp_body,
        grid=(n // _B2,),
        in_specs=[
            pl.BlockSpec((_B2, 128), lambda i: (i, 0)),
            whole((1, 128)), whole(v1.shape), whole((128, 1)), whole((1, 128)),
            whole(v2.shape), whole((1, 1)), whole((1, 1)),
        ],
        out_specs=pl.BlockSpec((1, _B2), lambda i: (0, i)),
        out_shape=jax.ShapeDtypeStruct((1, n), f32),
        compiler_params=pltpu.CompilerParams(
            dimension_semantics=("arbitrary",)),
    )(h0, b0.reshape(1, -1), v1, g1.reshape(-1, 1), b1.reshape(1, -1),
      v2, g2.reshape(1, 1), b2.reshape(1, 1))
    return out.reshape(n, 1)


# speedup vs baseline: 8.1688x; 8.1688x over previous
import jax
import jax.numpy as jnp
from jax import lax
from jax.experimental import pallas as pl
from jax.experimental.pallas import tpu as pltpu
from jax.experimental.pallas import tpu_sc as plsc

_EXPR = 80
_L = 64
_C = 32
_NPAD = 72
_NW = 32
_CHUNK = 32
_NSC = 10240
_B = 2048


def _premix_body(E_ref, flx_ref, fly_ref, flz_ref, v0_ref, g0_ref,
                 mx_ref, my_ref, mz_ref):
    f32 = jnp.float32
    v0 = v0_ref[...]
    W0 = v0 * (g0_ref[...] * lax.rsqrt(jnp.sum(v0 * v0, axis=1, keepdims=True)))
    E = E_ref[...]
    for a, (fl_ref, m_ref) in enumerate(
            ((flx_ref, mx_ref), (fly_ref, my_ref), (flz_ref, mz_ref))):
        tab = jnp.dot(fl_ref[...], E, preferred_element_type=f32)
        W0a = jnp.concatenate(
            [W0[:, _C * a:_C * a + _C],
             W0[:, 3 * _C + _C * a:3 * _C + _C * a + _C]], axis=1)
        Ma = lax.dot_general(tab, W0a, (((1,), (1,)), ((), ())),
                             preferred_element_type=f32)
        m_ref[0:_L, :] = Ma
        m_ref[_L:_NPAD, :] = jnp.zeros((_NPAD - _L, 128), f32)


def _sc_body(mx_hbm, my_hbm, mz_hbm, x_hbm, y_hbm, z_hbm, out_hbm,
             xb, yb, zb, rxl, rxr, ryl, ryr, rzl, rzr, h0b, sem):
    npts = x_hbm.shape[0]
    pw = npts // _NW
    nchunks = pw // _CHUNK
    wid = lax.axis_index("s") * 2 + lax.axis_index("c")
    base = wid * pw

    lane = lax.iota(jnp.int32, 16)
    dn = lax.GatherDimensionNumbers(
        offset_dims=(), collapsed_slice_dims=(0,), start_index_map=(0,))

    def splat(vec, j):
        idx = jnp.reshape(lane * 0 + j, (16, 1))
        return lax.gather(vec, idx, dn, (1,),
                          mode=lax.GatherScatterMode.PROMISE_IN_BOUNDS)

    axes = ((xb, mx_hbm, rxl, rxr), (yb, my_hbm, ryl, ryr),
            (zb, mz_hbm, rzl, rzr))

    def chunk_body(ci, carry):
        cbase = base + ci * _CHUNK
        pltpu.sync_copy(x_hbm.at[pl.ds(cbase, _CHUNK)], xb)
        pltpu.sync_copy(y_hbm.at[pl.ds(cbase, _CHUNK)], yb)
        pltpu.sync_copy(z_hbm.at[pl.ds(cbase, _CHUNK)], zb)

        ws = []
        handles = []
        for g in range(_CHUNK // 16):
            for buf, tab, rl_ref, rr_ref in axes:
                p = buf[pl.ds(g * 16, 16)]
                p = jnp.minimum(jnp.maximum(p, 0.0), 1.0) * (_L - 1.0)
                li = p.astype(jnp.int32)
                ws.append(p - li.astype(jnp.float32))
                dst_l = rl_ref.at[pl.ds(g * 16, 16), :]
                dst_r = rr_ref.at[pl.ds(g * 16, 16), :]
                handles.append(pltpu.async_copy(tab.at[li], dst_l, sem))
                handles.append(pltpu.async_copy(tab.at[li + 1], dst_r, sem))
        for h in handles:
            h.wait()

        for g in range(_CHUNK // 16):
            for j in range(16):
                pt = g * 16 + j
                acc = [None] * 8
                for a in range(3):
                    _, _, rl_ref, rr_ref = axes[a]
                    wv = splat(ws[g * 3 + a], j)
                    for k in range(8):
                        rl = rl_ref[pt, pl.ds(16 * k, 16)]
                        rr = rr_ref[pt, pl.ds(16 * k, 16)]
                        c = rl + wv * (rr - rl)
                        acc[k] = c if a == 0 else acc[k] + c
                hstart = pt * 128
                for k in range(8):
                    h0b[pl.ds(hstart + 16 * k, 16)] = acc[k]
        pltpu.sync_copy(h0b, out_hbm.at[pl.ds(cbase * 128, _CHUNK * 128)])
        return carry

    lax.fori_loop(0, nchunks, chunk_body, 0, unroll=False)


def _fused_body(E_ref, flx_ref, fly_ref, flz_ref,
                v0_ref, g0_ref, b0_ref, v1_ref, g1_ref, b1_ref,
                v2_ref, g2_ref, b2_ref, xyz_ref, o_ref,
                M_s, W1_s, W2_s):
    f32 = jnp.float32

    @pl.when(pl.program_id(0) == 0)
    def _init():
        v0 = v0_ref[...]
        W0 = v0 * (g0_ref[...] * lax.rsqrt(
            jnp.sum(v0 * v0, axis=1, keepdims=True)))
        E = E_ref[...]
        for a, fl_ref in enumerate((flx_ref, fly_ref, flz_ref)):
            tab = jnp.dot(fl_ref[...], E, preferred_element_type=f32)
            W0a = jnp.concatenate(
                [W0[:, _C * a:_C * a + _C],
                 W0[:, 3 * _C + _C * a:3 * _C + _C * a + _C]], axis=1)
            MaT = lax.dot_general(
                W0a, tab, (((1,), (1,)), ((), ())),
                preferred_element_type=f32)
            M_s[:, _L * a:_L * a + _L] = MaT
        v1 = v1_ref[...]
        W1_s[...] = v1 * (g1_ref[...] * lax.rsqrt(
            jnp.sum(v1 * v1, axis=1, keepdims=True)))
        v2 = v2_ref[...]
        W2_s[...] = v2 * (g2_ref[...] * lax.rsqrt(
            jnp.sum(v2 * v2, axis=1, keepdims=True)))

    p = jnp.clip(xyz_ref[...], 0.0, 1.0) * (_L - 1.0)
    iot = lax.broadcasted_iota(jnp.int32, (_L, _B), 0).astype(f32)
    hats = [jnp.maximum(1.0 - jnp.abs(p[a:a + 1, :] - iot), 0.0)
            for a in range(3)]
    Hall = jnp.concatenate(hats, axis=0)
    h = jnp.dot(M_s[...], Hall, preferred_element_type=f32) + b0_ref[...]
    h = jnp.maximum(h, 0.0)
    h = jnp.dot(W1_s[...], h, preferred_element_type=f32) + b1_ref[...]
    h = jnp.maximum(h, 0.0)
    o_ref[...] = jnp.dot(W2_s[...], h, preferred_element_type=f32) + b2_ref[...]


def _mlp_body(h0_ref, b0_ref, v1_ref, g1_ref, b1_ref, v2_ref, g2_ref, b2_ref,
              o_ref):
    f32 = jnp.float32
    v1 = v1_ref[...]
    W1 = v1 * (g1_ref[...] * lax.rsqrt(jnp.sum(v1 * v1, axis=1, keepdims=True)))
    v2 = v2_ref[...]
    W2 = v2 * (g2_ref[...] * lax.rsqrt(jnp.sum(v2 * v2, axis=1, keepdims=True)))
    a0 = jnp.maximum(h0_ref[...] + b0_ref[...], 0.0)
    h1 = lax.dot_general(a0, W1, (((1,), (1,)), ((), ())),
                         preferred_element_type=f32) + b1_ref[...]
    h1 = jnp.maximum(h1, 0.0)
    o_ref[...] = lax.dot_general(W2, h1, (((1,), (1,)), ((), ())),
                                 preferred_element_type=f32) + b2_ref[...]


@jax.jit
def kernel(expr, jaw_quat_weight, xyz, feat_lines_x, feat_lines_y,
           feat_lines_z, v0, g0, b0, v1, g1, b1, v2, g2, b2):
    f32 = jnp.float32
    n = xyz.shape[0]
    e = expr.reshape(-1)[:_EXPR]
    jw = jaw_quat_weight.reshape(-1)
    eye = jnp.eye(_C, dtype=f32)
    Ebs = (e[:, None, None] * eye).reshape(_EXPR * _C, _C)
    Ejw = (jw[:, None, None] * eye).reshape(jw.shape[0] * _C, _C)
    E = jnp.concatenate([
        jnp.concatenate([Ebs, jnp.zeros_like(Ebs)], axis=1),
        jnp.concatenate([jnp.zeros_like(Ejw), Ejw], axis=1)], axis=0)

    def lines2d(fl):
        return fl.transpose(1, 0, 2).reshape(_L, fl.shape[0] * _C)

    flx2, fly2, flz2 = map(lines2d, (feat_lines_x, feat_lines_y, feat_lines_z))
    xyzT = xyz.T
    whole = lambda shp: pl.BlockSpec(shp, lambda *_: (0,) * len(shp))

    mx, my, mz = pl.pallas_call(
        _premix_body,
        grid=(1,),
        in_specs=[whole(E.shape), whole((_L, 96 * _C)), whole((_L, 96 * _C)),
                  whole((_L, 96 * _C)), whole(v0.shape), whole((128, 1))],
        out_specs=[whole((_NPAD, 128))] * 3,
        out_shape=[jax.ShapeDtypeStruct((_NPAD, 128), f32)] * 3,
    )(E, flx2, fly2, flz2, v0, g0.reshape(-1, 1))

    mesh = plsc.VectorSubcoreMesh(core_axis_name="c", subcore_axis_name="s")
    sc = pl.kernel(
        _sc_body, mesh=mesh,
        out_type=jax.ShapeDtypeStruct((_NSC * 128,), f32),
        scratch_types=[
            pltpu.VMEM((_CHUNK,), f32),
            pltpu.VMEM((_CHUNK,), f32),
            pltpu.VMEM((_CHUNK,), f32),
            pltpu.VMEM((_CHUNK, 128), f32),
            pltpu.VMEM((_CHUNK, 128), f32),
            pltpu.VMEM((_CHUNK, 128), f32),
            pltpu.VMEM((_CHUNK, 128), f32),
            pltpu.VMEM((_CHUNK, 128), f32),
            pltpu.VMEM((_CHUNK, 128), f32),
            pltpu.VMEM((_CHUNK * 128,), f32),
            pltpu.SemaphoreType.DMA,
        ],
    )
    h0 = sc(mx, my, mz, xyzT[0, :_NSC], xyzT[1, :_NSC],
            xyzT[2, :_NSC]).reshape(_NSC, 128)

    ntc = n - _NSC
    out_tc = pl.pallas_call(
        _fused_body,
        grid=(ntc // _B,),
        in_specs=[
            whole(E.shape),
            whole((_L, 96 * _C)), whole((_L, 96 * _C)), whole((_L, 96 * _C)),
            whole(v0.shape), whole((v0.shape[0], 1)), whole((v0.shape[0], 1)),
            whole(v1.shape), whole((v1.shape[0], 1)), whole((v1.shape[0], 1)),
            whole(v2.shape), whole((1, 1)), whole((1, 1)),
            pl.BlockSpec((3, _B), lambda i: (0, i)),
        ],
        out_specs=pl.BlockSpec((1, _B), lambda i: (0, i)),
        out_shape=jax.ShapeDtypeStruct((1, ntc), f32),
        scratch_shapes=[
            pltpu.VMEM((128, 3 * _L), f32),
            pltpu.VMEM((128, 128), f32),
            pltpu.VMEM((1, 128), f32),
        ],
        compiler_params=pltpu.CompilerParams(
            dimension_semantics=("arbitrary",)),
    )(E, flx2, fly2, flz2,
      v0, g0.reshape(-1, 1), b0.reshape(-1, 1),
      v1, g1.reshape(-1, 1), b1.reshape(-1, 1),
      v2, g2.reshape(1, 1), b2.reshape(1, 1),
      xyzT[:, _NSC:])

    out_sc = pl.pallas_call(
        _mlp_body,
        grid=(_NSC // _B,),
        in_specs=[
            pl.BlockSpec((_B, 128), lambda i: (i, 0)),
            whole((1, 128)), whole(v1.shape), whole((128, 1)), whole((1, 128)),
            whole(v2.shape), whole((1, 1)), whole((1, 1)),
        ],
        out_specs=pl.BlockSpec((1, _B), lambda i: (0, i)),
        out_shape=jax.ShapeDtypeStruct((1, _NSC), f32),
        compiler_params=pltpu.CompilerParams(
            dimension_semantics=("arbitrary",)),
    )(h0, b0.reshape(1, -1), v1, g1.reshape(-1, 1), b1.reshape(1, -1),
      v2, g2.reshape(1, 1), b2.reshape(1, 1))

    return jnp.concatenate([out_sc, out_tc], axis=1).reshape(n, 1)


# speedup vs baseline: 9.4449x; 1.1562x over previous
import jax
import jax.numpy as jnp
from jax import lax
from jax.experimental import pallas as pl
from jax.experimental.pallas import tpu as pltpu
from jax.experimental.pallas import tpu_sc as plsc

_EXPR = 80
_L = 64
_C = 32
_NPAD = 72
_NW = 32
_CHUNK = 32
_NSC = 6144
_B = 2048


def _premix_body(E_ref, flx_ref, fly_ref, flz_ref, v0_ref, g0_ref,
                 mx_ref, my_ref, mz_ref):
    f32 = jnp.float32
    v0 = v0_ref[...]
    W0 = v0 * (g0_ref[...] * lax.rsqrt(jnp.sum(v0 * v0, axis=1, keepdims=True)))
    E = E_ref[...]
    for a, (fl_ref, m_ref) in enumerate(
            ((flx_ref, mx_ref), (fly_ref, my_ref), (flz_ref, mz_ref))):
        tab = jnp.dot(fl_ref[...], E, preferred_element_type=f32)
        W0a = jnp.concatenate(
            [W0[:, _C * a:_C * a + _C],
             W0[:, 3 * _C + _C * a:3 * _C + _C * a + _C]], axis=1)
        Ma = lax.dot_general(tab, W0a, (((1,), (1,)), ((), ())),
                             preferred_element_type=f32)
        m_ref[0:_L, :] = Ma
        m_ref[_L:_NPAD, :] = jnp.zeros((_NPAD - _L, 128), f32)


def _sc_body(mx_hbm, my_hbm, mz_hbm, x_hbm, y_hbm, z_hbm, out_hbm,
             xb, yb, zb, rxl, rxr, ryl, ryr, rzl, rzr, h0b, sem):
    npts = x_hbm.shape[0]
    pw = npts // _NW
    nchunks = pw // _CHUNK
    wid = lax.axis_index("s") * 2 + lax.axis_index("c")
    base = wid * pw

    lane = lax.iota(jnp.int32, 16)
    dn = lax.GatherDimensionNumbers(
        offset_dims=(), collapsed_slice_dims=(0,), start_index_map=(0,))

    def splat(vec, j):
        idx = jnp.reshape(lane * 0 + j, (16, 1))
        return lax.gather(vec, idx, dn, (1,),
                          mode=lax.GatherScatterMode.PROMISE_IN_BOUNDS)

    axes = ((xb, mx_hbm, rxl, rxr), (yb, my_hbm, ryl, ryr),
            (zb, mz_hbm, rzl, rzr))

    def chunk_body(ci, carry):
        cbase = base + ci * _CHUNK
        pltpu.sync_copy(x_hbm.at[pl.ds(cbase, _CHUNK)], xb)
        pltpu.sync_copy(y_hbm.at[pl.ds(cbase, _CHUNK)], yb)
        pltpu.sync_copy(z_hbm.at[pl.ds(cbase, _CHUNK)], zb)

        ws = []
        handles = []
        for g in range(_CHUNK // 16):
            for buf, tab, rl_ref, rr_ref in axes:
                p = buf[pl.ds(g * 16, 16)]
                p = jnp.minimum(jnp.maximum(p, 0.0), 1.0) * (_L - 1.0)
                li = p.astype(jnp.int32)
                ws.append(p - li.astype(jnp.float32))
                dst_l = rl_ref.at[pl.ds(g * 16, 16), :]
                dst_r = rr_ref.at[pl.ds(g * 16, 16), :]
                handles.append(pltpu.async_copy(tab.at[li], dst_l, sem))
                handles.append(pltpu.async_copy(tab.at[li + 1], dst_r, sem))
        for h in handles:
            h.wait()

        for g in range(_CHUNK // 16):
            for j in range(16):
                pt = g * 16 + j
                acc = [None] * 8
                for a in range(3):
                    _, _, rl_ref, rr_ref = axes[a]
                    wv = splat(ws[g * 3 + a], j)
                    for k in range(8):
                        rl = rl_ref[pt, pl.ds(16 * k, 16)]
                        rr = rr_ref[pt, pl.ds(16 * k, 16)]
                        c = rl + wv * (rr - rl)
                        acc[k] = c if a == 0 else acc[k] + c
                hstart = pt * 128
                for k in range(8):
                    h0b[pl.ds(hstart + 16 * k, 16)] = acc[k]
        pltpu.sync_copy(h0b, out_hbm.at[pl.ds(cbase * 128, _CHUNK * 128)])
        return carry

    lax.fori_loop(0, nchunks, chunk_body, 0, unroll=False)


def _fused_body(E_ref, flx_ref, fly_ref, flz_ref,
                v0_ref, g0_ref, b0_ref, v1_ref, g1_ref, b1_ref,
                v2_ref, g2_ref, b2_ref, xyz_ref, o_ref,
                M_s, W1_s, W2_s):
    f32 = jnp.float32

    @pl.when(pl.program_id(0) == 0)
    def _init():
        v0 = v0_ref[...]
        W0 = v0 * (g0_ref[...] * lax.rsqrt(
            jnp.sum(v0 * v0, axis=1, keepdims=True)))
        E = E_ref[...]
        for a, fl_ref in enumerate((flx_ref, fly_ref, flz_ref)):
            tab = jnp.dot(fl_ref[...], E, preferred_element_type=f32)
            W0a = jnp.concatenate(
                [W0[:, _C * a:_C * a + _C],
                 W0[:, 3 * _C + _C * a:3 * _C + _C * a + _C]], axis=1)
            MaT = lax.dot_general(
                W0a, tab, (((1,), (1,)), ((), ())),
                preferred_element_type=f32)
            M_s[:, _L * a:_L * a + _L] = MaT
        v1 = v1_ref[...]
        W1_s[...] = v1 * (g1_ref[...] * lax.rsqrt(
            jnp.sum(v1 * v1, axis=1, keepdims=True)))
        v2 = v2_ref[...]
        W2_s[...] = v2 * (g2_ref[...] * lax.rsqrt(
            jnp.sum(v2 * v2, axis=1, keepdims=True)))

    p = jnp.clip(xyz_ref[...], 0.0, 1.0) * (_L - 1.0)
    iot = lax.broadcasted_iota(jnp.int32, (_L, _B), 0).astype(f32)
    hats = [jnp.maximum(1.0 - jnp.abs(p[a:a + 1, :] - iot), 0.0)
            for a in range(3)]
    Hall = jnp.concatenate(hats, axis=0)
    h = jnp.dot(M_s[...], Hall, preferred_element_type=f32) + b0_ref[...]
    h = jnp.maximum(h, 0.0)
    h = jnp.dot(W1_s[...], h, preferred_element_type=f32) + b1_ref[...]
    h = jnp.maximum(h, 0.0)
    o_ref[...] = jnp.dot(W2_s[...], h, preferred_element_type=f32) + b2_ref[...]


def _mlp_body(h0_ref, b0_ref, v1_ref, g1_ref, b1_ref, v2_ref, g2_ref, b2_ref,
              o_ref):
    f32 = jnp.float32
    v1 = v1_ref[...]
    W1 = v1 * (g1_ref[...] * lax.rsqrt(jnp.sum(v1 * v1, axis=1, keepdims=True)))
    v2 = v2_ref[...]
    W2 = v2 * (g2_ref[...] * lax.rsqrt(jnp.sum(v2 * v2, axis=1, keepdims=True)))
    a0 = jnp.maximum(h0_ref[...] + b0_ref[...], 0.0)
    h1 = lax.dot_general(a0, W1, (((1,), (1,)), ((), ())),
                         preferred_element_type=f32) + b1_ref[...]
    h1 = jnp.maximum(h1, 0.0)
    o_ref[...] = lax.dot_general(W2, h1, (((1,), (1,)), ((), ())),
                                 preferred_element_type=f32) + b2_ref[...]


@jax.jit
def kernel(expr, jaw_quat_weight, xyz, feat_lines_x, feat_lines_y,
           feat_lines_z, v0, g0, b0, v1, g1, b1, v2, g2, b2):
    f32 = jnp.float32
    n = xyz.shape[0]
    e = expr.reshape(-1)[:_EXPR]
    jw = jaw_quat_weight.reshape(-1)
    eye = jnp.eye(_C, dtype=f32)
    Ebs = (e[:, None, None] * eye).reshape(_EXPR * _C, _C)
    Ejw = (jw[:, None, None] * eye).reshape(jw.shape[0] * _C, _C)
    E = jnp.concatenate([
        jnp.concatenate([Ebs, jnp.zeros_like(Ebs)], axis=1),
        jnp.concatenate([jnp.zeros_like(Ejw), Ejw], axis=1)], axis=0)

    def lines2d(fl):
        return fl.transpose(1, 0, 2).reshape(_L, fl.shape[0] * _C)

    flx2, fly2, flz2 = map(lines2d, (feat_lines_x, feat_lines_y, feat_lines_z))
    xyzT = xyz.T
    whole = lambda shp: pl.BlockSpec(shp, lambda *_: (0,) * len(shp))

    mx, my, mz = pl.pallas_call(
        _premix_body,
        grid=(1,),
        in_specs=[whole(E.shape), whole((_L, 96 * _C)), whole((_L, 96 * _C)),
                  whole((_L, 96 * _C)), whole(v0.shape), whole((128, 1))],
        out_specs=[whole((_NPAD, 128))] * 3,
        out_shape=[jax.ShapeDtypeStruct((_NPAD, 128), f32)] * 3,
    )(E, flx2, fly2, flz2, v0, g0.reshape(-1, 1))

    mesh = plsc.VectorSubcoreMesh(core_axis_name="c", subcore_axis_name="s")
    sc = pl.kernel(
        _sc_body, mesh=mesh,
        out_type=jax.ShapeDtypeStruct((_NSC * 128,), f32),
        scratch_types=[
            pltpu.VMEM((_CHUNK,), f32),
            pltpu.VMEM((_CHUNK,), f32),
            pltpu.VMEM((_CHUNK,), f32),
            pltpu.VMEM((_CHUNK, 128), f32),
            pltpu.VMEM((_CHUNK, 128), f32),
            pltpu.VMEM((_CHUNK, 128), f32),
            pltpu.VMEM((_CHUNK, 128), f32),
            pltpu.VMEM((_CHUNK, 128), f32),
            pltpu.VMEM((_CHUNK, 128), f32),
            pltpu.VMEM((_CHUNK * 128,), f32),
            pltpu.SemaphoreType.DMA,
        ],
    )
    h0 = sc(mx, my, mz, xyzT[0, :_NSC], xyzT[1, :_NSC],
            xyzT[2, :_NSC]).reshape(_NSC, 128)

    ntc = n - _NSC
    out_tc = pl.pallas_call(
        _fused_body,
        grid=(ntc // _B,),
        in_specs=[
            whole(E.shape),
            whole((_L, 96 * _C)), whole((_L, 96 * _C)), whole((_L, 96 * _C)),
            whole(v0.shape), whole((v0.shape[0], 1)), whole((v0.shape[0], 1)),
            whole(v1.shape), whole((v1.shape[0], 1)), whole((v1.shape[0], 1)),
            whole(v2.shape), whole((1, 1)), whole((1, 1)),
            pl.BlockSpec((3, _B), lambda i: (0, i)),
        ],
        out_specs=pl.BlockSpec((1, _B), lambda i: (0, i)),
        out_shape=jax.ShapeDtypeStruct((1, ntc), f32),
        scratch_shapes=[
            pltpu.VMEM((128, 3 * _L), f32),
            pltpu.VMEM((128, 128), f32),
            pltpu.VMEM((1, 128), f32),
        ],
        compiler_params=pltpu.CompilerParams(
            dimension_semantics=("arbitrary",)),
    )(E, flx2, fly2, flz2,
      v0, g0.reshape(-1, 1), b0.reshape(-1, 1),
      v1, g1.reshape(-1, 1), b1.reshape(-1, 1),
      v2, g2.reshape(1, 1), b2.reshape(1, 1),
      xyzT[:, _NSC:])

    out_sc = pl.pallas_call(
        _mlp_body,
        grid=(_NSC // _B,),
        in_specs=[
            pl.BlockSpec((_B, 128), lambda i: (i, 0)),
            whole((1, 128)), whole(v1.shape), whole((128, 1)), whole((1, 128)),
            whole(v2.shape), whole((1, 1)), whole((1, 1)),
        ],
        out_specs=pl.BlockSpec((1, _B), lambda i: (0, i)),
        out_shape=jax.ShapeDtypeStruct((1, _NSC), f32),
        compiler_params=pltpu.CompilerParams(
            dimension_semantics=("arbitrary",)),
    )(h0, b0.reshape(1, -1), v1, g1.reshape(-1, 1), b1.reshape(1, -1),
      v2, g2.reshape(1, 1), b2.reshape(1, 1))

    return jnp.concatenate([out_sc, out_tc], axis=1).reshape(n, 1)


# speedup vs baseline: 9.5234x; 1.0083x over previous
import jax
import jax.numpy as jnp
from jax import lax
from jax.experimental import pallas as pl
from jax.experimental.pallas import tpu as pltpu
from jax.experimental.pallas import tpu_sc as plsc

_EXPR = 80
_L = 64
_C = 32
_NPAD = 72
_NW = 32
_CHUNK = 32
_NSC = 4096
_B = 2048


def _premix_body(E_ref, flx_ref, fly_ref, flz_ref, v0_ref, g0_ref,
                 mx_ref, my_ref, mz_ref):
    f32 = jnp.float32
    v0 = v0_ref[...]
    W0 = v0 * (g0_ref[...] * lax.rsqrt(jnp.sum(v0 * v0, axis=1, keepdims=True)))
    E = E_ref[...]
    for a, (fl_ref, m_ref) in enumerate(
            ((flx_ref, mx_ref), (fly_ref, my_ref), (flz_ref, mz_ref))):
        tab = jnp.dot(fl_ref[...], E, preferred_element_type=f32)
        W0a = jnp.concatenate(
            [W0[:, _C * a:_C * a + _C],
             W0[:, 3 * _C + _C * a:3 * _C + _C * a + _C]], axis=1)
        Ma = lax.dot_general(tab, W0a, (((1,), (1,)), ((), ())),
                             preferred_element_type=f32)
        m_ref[0:_L, :] = Ma
        m_ref[_L:_NPAD, :] = jnp.zeros((_NPAD - _L, 128), f32)


def _sc_body(mx_hbm, my_hbm, mz_hbm, x_hbm, y_hbm, z_hbm, out_hbm,
             xb, yb, zb, rxl, rxr, ryl, ryr, rzl, rzr, h0b, sem):
    npts = x_hbm.shape[0]
    pw = npts // _NW
    nchunks = pw // _CHUNK
    wid = lax.axis_index("s") * 2 + lax.axis_index("c")
    base = wid * pw

    lane = lax.iota(jnp.int32, 16)
    dn = lax.GatherDimensionNumbers(
        offset_dims=(), collapsed_slice_dims=(0,), start_index_map=(0,))

    def splat(vec, j):
        idx = jnp.reshape(lane * 0 + j, (16, 1))
        return lax.gather(vec, idx, dn, (1,),
                          mode=lax.GatherScatterMode.PROMISE_IN_BOUNDS)

    axes = ((xb, mx_hbm, rxl, rxr), (yb, my_hbm, ryl, ryr),
            (zb, mz_hbm, rzl, rzr))

    def chunk_body(ci, carry):
        cbase = base + ci * _CHUNK
        pltpu.sync_copy(x_hbm.at[pl.ds(cbase, _CHUNK)], xb)
        pltpu.sync_copy(y_hbm.at[pl.ds(cbase, _CHUNK)], yb)
        pltpu.sync_copy(z_hbm.at[pl.ds(cbase, _CHUNK)], zb)

        ws = []
        handles = []
        for g in range(_CHUNK // 16):
            for buf, tab, rl_ref, rr_ref in axes:
                p = buf[pl.ds(g * 16, 16)]
                p = jnp.minimum(jnp.maximum(p, 0.0), 1.0) * (_L - 1.0)
                li = p.astype(jnp.int32)
                ws.append(p - li.astype(jnp.float32))
                dst_l = rl_ref.at[pl.ds(g * 16, 16), :]
                dst_r = rr_ref.at[pl.ds(g * 16, 16), :]
                handles.append(pltpu.async_copy(tab.at[li], dst_l, sem))
                handles.append(pltpu.async_copy(tab.at[li + 1], dst_r, sem))
        for h in handles:
            h.wait()

        for g in range(_CHUNK // 16):
            for j in range(16):
                pt = g * 16 + j
                acc = [None] * 8
                for a in range(3):
                    _, _, rl_ref, rr_ref = axes[a]
                    wv = splat(ws[g * 3 + a], j)
                    for k in range(8):
                        rl = rl_ref[pt, pl.ds(16 * k, 16)]
                        rr = rr_ref[pt, pl.ds(16 * k, 16)]
                        c = rl + wv * (rr - rl)
                        acc[k] = c if a == 0 else acc[k] + c
                hstart = pt * 128
                for k in range(8):
                    h0b[pl.ds(hstart + 16 * k, 16)] = acc[k]
        pltpu.sync_copy(h0b, out_hbm.at[pl.ds(cbase * 128, _CHUNK * 128)])
        return carry

    lax.fori_loop(0, nchunks, chunk_body, 0, unroll=False)


def _fused_body(E_ref, flx_ref, fly_ref, flz_ref,
                v0_ref, g0_ref, b0_ref, v1_ref, g1_ref, b1_ref,
                v2_ref, g2_ref, b2_ref, xyz_ref, o_ref,
                M_s, W1_s, W2_s):
    f32 = jnp.float32

    @pl.when(pl.program_id(0) == 0)
    def _init():
        v0 = v0_ref[...]
        W0 = v0 * (g0_ref[...] * lax.rsqrt(
            jnp.sum(v0 * v0, axis=1, keepdims=True)))
        E = E_ref[...]
        for a, fl_ref in enumerate((flx_ref, fly_ref, flz_ref)):
            tab = jnp.dot(fl_ref[...], E, preferred_element_type=f32)
            W0a = jnp.concatenate(
                [W0[:, _C * a:_C * a + _C],
                 W0[:, 3 * _C + _C * a:3 * _C + _C * a + _C]], axis=1)
            MaT = lax.dot_general(
                W0a, tab, (((1,), (1,)), ((), ())),
                preferred_element_type=f32)
            M_s[:, _L * a:_L * a + _L] = MaT
        v1 = v1_ref[...]
        W1_s[...] = v1 * (g1_ref[...] * lax.rsqrt(
            jnp.sum(v1 * v1, axis=1, keepdims=True)))
        v2 = v2_ref[...]
        W2_s[...] = v2 * (g2_ref[...] * lax.rsqrt(
            jnp.sum(v2 * v2, axis=1, keepdims=True)))

    p = jnp.clip(xyz_ref[...], 0.0, 1.0) * (_L - 1.0)
    iot = lax.broadcasted_iota(jnp.int32, (_L, _B), 0).astype(f32)
    hats = [jnp.maximum(1.0 - jnp.abs(p[a:a + 1, :] - iot), 0.0)
            for a in range(3)]
    Hall = jnp.concatenate(hats, axis=0)
    h = jnp.dot(M_s[...], Hall, preferred_element_type=f32) + b0_ref[...]
    h = jnp.maximum(h, 0.0)
    h = jnp.dot(W1_s[...], h, preferred_element_type=f32) + b1_ref[...]
    h = jnp.maximum(h, 0.0)
    o_ref[...] = jnp.dot(W2_s[...], h, preferred_element_type=f32) + b2_ref[...]


def _mlp_body(h0_ref, b0_ref, v1_ref, g1_ref, b1_ref, v2_ref, g2_ref, b2_ref,
              o_ref):
    f32 = jnp.float32
    v1 = v1_ref[...]
    W1 = v1 * (g1_ref[...] * lax.rsqrt(jnp.sum(v1 * v1, axis=1, keepdims=True)))
    v2 = v2_ref[...]
    W2 = v2 * (g2_ref[...] * lax.rsqrt(jnp.sum(v2 * v2, axis=1, keepdims=True)))
    a0 = jnp.maximum(h0_ref[...] + b0_ref[...], 0.0)
    h1 = lax.dot_general(a0, W1, (((1,), (1,)), ((), ())),
                         preferred_element_type=f32) + b1_ref[...]
    h1 = jnp.maximum(h1, 0.0)
    o_ref[...] = lax.dot_general(W2, h1, (((1,), (1,)), ((), ())),
                                 preferred_element_type=f32) + b2_ref[...]


@jax.jit
def kernel(expr, jaw_quat_weight, xyz, feat_lines_x, feat_lines_y,
           feat_lines_z, v0, g0, b0, v1, g1, b1, v2, g2, b2):
    f32 = jnp.float32
    n = xyz.shape[0]
    e = expr.reshape(-1)[:_EXPR]
    jw = jaw_quat_weight.reshape(-1)
    eye = jnp.eye(_C, dtype=f32)
    Ebs = (e[:, None, None] * eye).reshape(_EXPR * _C, _C)
    Ejw = (jw[:, None, None] * eye).reshape(jw.shape[0] * _C, _C)
    E = jnp.concatenate([
        jnp.concatenate([Ebs, jnp.zeros_like(Ebs)], axis=1),
        jnp.concatenate([jnp.zeros_like(Ejw), Ejw], axis=1)], axis=0)

    def lines2d(fl):
        return fl.transpose(1, 0, 2).reshape(_L, fl.shape[0] * _C)

    flx2, fly2, flz2 = map(lines2d, (feat_lines_x, feat_lines_y, feat_lines_z))
    xyzT = xyz.T
    whole = lambda shp: pl.BlockSpec(shp, lambda *_: (0,) * len(shp))

    mx, my, mz = pl.pallas_call(
        _premix_body,
        grid=(1,),
        in_specs=[whole(E.shape), whole((_L, 96 * _C)), whole((_L, 96 * _C)),
                  whole((_L, 96 * _C)), whole(v0.shape), whole((128, 1))],
        out_specs=[whole((_NPAD, 128))] * 3,
        out_shape=[jax.ShapeDtypeStruct((_NPAD, 128), f32)] * 3,
    )(E, flx2, fly2, flz2, v0, g0.reshape(-1, 1))

    mesh = plsc.VectorSubcoreMesh(core_axis_name="c", subcore_axis_name="s")
    sc = pl.kernel(
        _sc_body, mesh=mesh,
        out_type=jax.ShapeDtypeStruct((_NSC * 128,), f32),
        scratch_types=[
            pltpu.VMEM((_CHUNK,), f32),
            pltpu.VMEM((_CHUNK,), f32),
            pltpu.VMEM((_CHUNK,), f32),
            pltpu.VMEM((_CHUNK, 128), f32),
            pltpu.VMEM((_CHUNK, 128), f32),
            pltpu.VMEM((_CHUNK, 128), f32),
            pltpu.VMEM((_CHUNK, 128), f32),
            pltpu.VMEM((_CHUNK, 128), f32),
            pltpu.VMEM((_CHUNK, 128), f32),
            pltpu.VMEM((_CHUNK * 128,), f32),
            pltpu.SemaphoreType.DMA,
        ],
    )
    h0 = sc(mx, my, mz, xyzT[0, :_NSC], xyzT[1, :_NSC],
            xyzT[2, :_NSC]).reshape(_NSC, 128)

    ntc = n - _NSC
    out_tc = pl.pallas_call(
        _fused_body,
        grid=(ntc // _B,),
        in_specs=[
            whole(E.shape),
            whole((_L, 96 * _C)), whole((_L, 96 * _C)), whole((_L, 96 * _C)),
            whole(v0.shape), whole((v0.shape[0], 1)), whole((v0.shape[0], 1)),
            whole(v1.shape), whole((v1.shape[0], 1)), whole((v1.shape[0], 1)),
            whole(v2.shape), whole((1, 1)), whole((1, 1)),
            pl.BlockSpec((3, _B), lambda i: (0, i)),
        ],
        out_specs=pl.BlockSpec((1, _B), lambda i: (0, i)),
        out_shape=jax.ShapeDtypeStruct((1, ntc), f32),
        scratch_shapes=[
            pltpu.VMEM((128, 3 * _L), f32),
            pltpu.VMEM((128, 128), f32),
            pltpu.VMEM((1, 128), f32),
        ],
        compiler_params=pltpu.CompilerParams(
            dimension_semantics=("arbitrary",)),
    )(E, flx2, fly2, flz2,
      v0, g0.reshape(-1, 1), b0.reshape(-1, 1),
      v1, g1.reshape(-1, 1), b1.reshape(-1, 1),
      v2, g2.reshape(1, 1), b2.reshape(1, 1),
      xyzT[:, _NSC:])

    out_sc = pl.pallas_call(
        _mlp_body,
        grid=(_NSC // _B,),
        in_specs=[
            pl.BlockSpec((_B, 128), lambda i: (i, 0)),
            whole((1, 128)), whole(v1.shape), whole((128, 1)), whole((1, 128)),
            whole(v2.shape), whole((1, 1)), whole((1, 1)),
        ],
        out_specs=pl.BlockSpec((1, _B), lambda i: (0, i)),
        out_shape=jax.ShapeDtypeStruct((1, _NSC), f32),
        compiler_params=pltpu.CompilerParams(
            dimension_semantics=("arbitrary",)),
    )(h0, b0.reshape(1, -1), v1, g1.reshape(-1, 1), b1.reshape(1, -1),
      v2, g2.reshape(1, 1), b2.reshape(1, 1))

    return jnp.concatenate([out_sc, out_tc], axis=1).reshape(n, 1)
